# baseline (speedup 1.0000x reference)
import functools

import jax
import jax.numpy as jnp
from jax import lax
from jax.experimental import pallas as pl
from jax.experimental.pallas import tpu as pltpu
from jax.experimental.pallas import tpu_sc as plsc

B, L, H = 4096, 50, 768
LP = 56
NC, NS = 2, 16
NW = NC * NS
BPW = B // NW
K = 8
CPE = LP // K
CH = BPW * CPE
D = 8
LA = 4
VPR = H // 16
REM = L - (CPE - 1) * K

assert CH % D == 0 and LA < D

_mesh = plsc.VectorSubcoreMesh(core_axis_name="c", subcore_axis_name="s")


@functools.partial(
    pl.kernel,
    mesh=_mesh,
    out_type=jax.ShapeDtypeStruct((B, L, H), jnp.float32),
    scratch_types=[
        pltpu.VMEM((CH * K,), jnp.int32),
        pltpu.VMEM((CH * K,), jnp.int32),
        [pltpu.VMEM((K, H), jnp.float32) for _ in range(D)],
        [pltpu.VMEM((K, H), jnp.float32) for _ in range(D)],
        [pltpu.SemaphoreType.DMA for _ in range(D)],
        [pltpu.SemaphoreType.DMA for _ in range(D)],
        [pltpu.SemaphoreType.DMA for _ in range(D)],
    ],
)
def _embed(ids_hbm, flg_hbm, itab_hbm, ftab_hbm, out_hbm,
           ids_v, flg_v, irows, frows, sem_i, sem_f, sem_o):
    wid = lax.axis_index("s") * NC + lax.axis_index("c")
    ebase = wid * BPW
    pltpu.sync_copy(ids_hbm.at[pl.ds(wid * CH * K, CH * K)], ids_v)
    pltpu.sync_copy(flg_hbm.at[pl.ds(wid * CH * K, CH * K)], flg_v)

    def fire(c, b):
        off = c * K
        pltpu.async_copy(itab_hbm.at[ids_v.at[pl.ds(off, K)]], irows[b], sem_i[b])
        pltpu.async_copy(ftab_hbm.at[flg_v.at[pl.ds(off, K)]], frows[b], sem_f[b])

    def drain_gather(b):
        pltpu.make_async_copy(itab_hbm.at[ids_v.at[pl.ds(0, K)]], irows[b], sem_i[b]).wait()
        pltpu.make_async_copy(ftab_hbm.at[flg_v.at[pl.ds(0, K)]], frows[b], sem_f[b]).wait()

    def fire_out(e, j, b):
        @pl.when(j < CPE - 1)
        def _():
            pltpu.async_copy(irows[b], out_hbm.at[e, pl.ds(j * K, K)], sem_o[b])

        @pl.when(j == CPE - 1)
        def _():
            pltpu.async_copy(irows[b].at[pl.ds(0, REM)],
                             out_hbm.at[e, pl.ds((CPE - 1) * K, REM)], sem_o[b])

    def drain_out(j, b):
        @pl.when(j < CPE - 1)
        def _():
            pltpu.make_async_copy(irows[b], out_hbm.at[0, pl.ds(0, K)], sem_o[b]).wait()

        @pl.when(j == CPE - 1)
        def _():
            pltpu.make_async_copy(irows[b].at[pl.ds(0, REM)],
                                  out_hbm.at[0, pl.ds(0, REM)], sem_o[b]).wait()

    def drain_out_static(c, b):
        if c % CPE < CPE - 1:
            pltpu.make_async_copy(irows[b], out_hbm.at[0, pl.ds(0, K)], sem_o[b]).wait()
        else:
            pltpu.make_async_copy(irows[b].at[pl.ds(0, REM)],
                                  out_hbm.at[0, pl.ds(0, REM)], sem_o[b]).wait()

    def bump(e, j):
        nj = j + 1
        wrap = nj == CPE
        return e + wrap.astype(jnp.int32), lax.select(wrap, jnp.int32(0), nj)

    for c0 in range(LA):
        fire(c0, c0)

    def outer(g, carry):
        e_c, j_c, j_d = carry
        for b in range(D):
            c = g * D + b
            b2 = (b + LA) % D

            @pl.when(c + LA < CH)
            def _():
                @pl.when(c >= D - LA)
                def _():
                    drain_out(j_d, b2)
                fire(c + LA, b2)

            drain_gather(b)

            def row(r, rc):
                for v in range(VPR):
                    sl = pl.ds(v * 16, 16)
                    irows[b][r, sl] = irows[b][r, sl] + frows[b][r, sl]
                return rc

            lax.fori_loop(0, K, row, 0)
            fire_out(ebase + e_c, j_c, b)
            e_c, j_c = bump(e_c, j_c)
            _, j_d = bump(jnp.int32(0), j_d)
        return (e_c, j_c, j_d)

    lax.fori_loop(0, CH // D, outer,
                  (jnp.int32(0), jnp.int32(0), jnp.int32(CPE - (D - LA))))

    for c0 in range(CH + LA - D, CH):
        drain_out_static(c0, c0 % D)


def kernel(item_ids, flags, item_table, flag_table):
    ids = jnp.pad(item_ids.astype(jnp.int32), ((0, 0), (0, LP - L))).reshape(B * LP)
    flg = jnp.pad(flags.astype(jnp.int32), ((0, 0), (0, LP - L))).reshape(B * LP)
    return _embed(ids, flg, item_table, flag_table)


# speedup vs baseline: 2.3358x; 2.3358x over previous
import functools

import jax
import jax.numpy as jnp
from jax import lax
from jax.experimental import pallas as pl
from jax.experimental.pallas import tpu as pltpu
from jax.experimental.pallas import tpu_sc as plsc

B, L, H = 4096, 50, 768
LP = 56
NC, NS = 2, 16
NW = NC * NS
BPW = B // NW
K = 8
CPE = LP // K
CH = BPW * CPE
D = 8
LA = 4
VPR = H // 16
REM = L - (CPE - 1) * K

assert CH % D == 0 and LA < D

_mesh = plsc.VectorSubcoreMesh(core_axis_name="c", subcore_axis_name="s")


@functools.partial(
    pl.kernel,
    mesh=_mesh,
    out_type=jax.ShapeDtypeStruct((B, L, H), jnp.float32),
    scratch_types=[
        pltpu.VMEM((CH * K,), jnp.int32),
        pltpu.VMEM((CH * K,), jnp.int32),
        [pltpu.VMEM((K, H), jnp.float32) for _ in range(D)],
        [pltpu.VMEM((K, H), jnp.float32) for _ in range(D)],
        [pltpu.SemaphoreType.DMA for _ in range(D)],
        [pltpu.SemaphoreType.DMA for _ in range(D)],
        [pltpu.SemaphoreType.DMA for _ in range(D)],
    ],
)
def _embed(ids_hbm, flg_hbm, itab_hbm, ftab_hbm, out_hbm,
           ids_v, flg_v, irows, frows, sem_i, sem_f, sem_o):
    wid = lax.axis_index("s") * NC + lax.axis_index("c")
    ebase = wid * BPW
    pltpu.sync_copy(ids_hbm.at[pl.ds(wid * CH * K, CH * K)], ids_v)
    pltpu.sync_copy(flg_hbm.at[pl.ds(wid * CH * K, CH * K)], flg_v)

    def fire(c, b):
        off = c * K
        pltpu.async_copy(itab_hbm.at[ids_v.at[pl.ds(off, K)]], irows[b], sem_i[b])
        pltpu.async_copy(ftab_hbm.at[flg_v.at[pl.ds(off, K)]], frows[b], sem_f[b])

    def drain_gather(b):
        pltpu.make_async_copy(itab_hbm.at[ids_v.at[pl.ds(0, K)]], irows[b], sem_i[b]).wait()
        pltpu.make_async_copy(ftab_hbm.at[flg_v.at[pl.ds(0, K)]], frows[b], sem_f[b]).wait()

    def fire_out(e, j, b):
        @pl.when(j < CPE - 1)
        def _():
            pltpu.async_copy(irows[b], out_hbm.at[e, pl.ds(j * K, K)], sem_o[b])

        @pl.when(j == CPE - 1)
        def _():
            pltpu.async_copy(irows[b].at[pl.ds(0, REM)],
                             out_hbm.at[e, pl.ds((CPE - 1) * K, REM)], sem_o[b])

    def drain_out(j, b):
        @pl.when(j < CPE - 1)
        def _():
            pltpu.make_async_copy(irows[b], out_hbm.at[0, pl.ds(0, K)], sem_o[b]).wait()

        @pl.when(j == CPE - 1)
        def _():
            pltpu.make_async_copy(irows[b].at[pl.ds(0, REM)],
                                  out_hbm.at[0, pl.ds(0, REM)], sem_o[b]).wait()

    def drain_out_static(c, b):
        if c % CPE < CPE - 1:
            pltpu.make_async_copy(irows[b], out_hbm.at[0, pl.ds(0, K)], sem_o[b]).wait()
        else:
            pltpu.make_async_copy(irows[b].at[pl.ds(0, REM)],
                                  out_hbm.at[0, pl.ds(0, REM)], sem_o[b]).wait()

    def bump(e, j):
        nj = j + 1
        wrap = nj == CPE
        return e + wrap.astype(jnp.int32), lax.select(wrap, jnp.int32(0), nj)

    for c0 in range(LA):
        fire(c0, c0)

    def outer(g, carry):
        e_c, j_c, j_d = carry
        for b in range(D):
            c = g * D + b
            b2 = (b + LA) % D

            @pl.when(c + LA < CH)
            def _():
                @pl.when(c >= D - LA)
                def _():
                    drain_out(j_d, b2)
                fire(c + LA, b2)

            drain_gather(b)

            def row(r, rc):
                for v in range(VPR):
                    sl = pl.ds(v * 16, 16)
                    irows[b][r, sl] = irows[b][r, sl] + frows[b][r, sl]
                return rc

            lax.fori_loop(0, K, row, 0)
            fire_out(ebase + e_c, j_c, b)
            e_c, j_c = bump(e_c, j_c)
            _, j_d = bump(jnp.int32(0), j_d)
        return (e_c, j_c, j_d)

    lax.fori_loop(0, CH // D, outer,
                  (jnp.int32(0), jnp.int32(0), jnp.int32(CPE - (D - LA))))

    for c0 in range(CH + LA - D, CH):
        drain_out_static(c0, c0 % D)


def kernel(item_ids, flags, item_table, flag_table):
    ids = jnp.pad(item_ids.astype(jnp.int32), ((0, 0), (0, LP - L)), mode='wrap').reshape(B * LP)
    flg = jnp.pad(flags.astype(jnp.int32), ((0, 0), (0, LP - L)), mode='wrap').reshape(B * LP)
    return _embed(ids, flg, item_table, flag_table)
